# baseline (speedup 1.0000x reference)
import jax
import jax.numpy as jnp
from jax.experimental import pallas as pl
from jax.experimental.pallas import tpu as pltpu

_E = 8
_DIM = 96
_HID = 384
_OUT = 96
_B = 2
_N = 1024
_NT = _B * _N
_BK = 256
_K = _NT // _BK

_CT = (((1,), (1,)), ((), ()))


def _row_to_col(row):
    i = jax.lax.broadcasted_iota(jnp.int32, (_E, _E), 0)
    j = jax.lax.broadcasted_iota(jnp.int32, (_E, _E), 1)
    b = jnp.broadcast_to(row, (_E, _E))
    return jnp.sum(jnp.where(i == j, b, 0.0), axis=1, keepdims=True)


def _shift_down(a):
    return jnp.concatenate([jnp.zeros((1, _HID), jnp.float32), a[:-1]], axis=0)


def _shift_up(a):
    return jnp.concatenate([a[1:], jnp.zeros((1, _HID), jnp.float32)], axis=0)


def _moe_kernel(x_ref, wg_ref, wgb_ref, fc1_ref, fc1b_ref, dw_ref,
                dwb_ref, fc2_ref, fc2b_ref, out_ref,
                zblk, pres, p0s, gsc, ys, sem):
    k = pl.program_id(0)
    b = k // (_K // _B)

    @pl.when(k == 0)
    def _init():
        zblk[...] = jnp.zeros((_BK, _OUT), jnp.float32)
        pres[...] = jnp.zeros((_B, _E), jnp.float32)

    pltpu.make_async_copy(zblk, out_ref.at[pl.ds(k * _BK, _BK), :],
                          sem.at[k]).start()

    x = x_ref[...]
    logits = jax.lax.dot_general(x, wg_ref[...], _CT,
                                 preferred_element_type=jnp.float32)
    logits = logits + wgb_ref[...]
    mx = jnp.max(logits, axis=1, keepdims=True)
    iota_e = jax.lax.broadcasted_iota(jnp.int32, (_BK, _E), 1)
    idx = jnp.min(jnp.where(logits == mx, iota_e, _E), axis=1, keepdims=True)
    onehot = jnp.where(iota_e == idx, 1.0, 0.0)
    presrow = jnp.max(onehot, axis=0, keepdims=True)
    rowsel = jax.lax.broadcasted_iota(jnp.int32, (_B, _E), 0) == b
    pres[...] = jnp.maximum(pres[...], jnp.where(rowsel, presrow, 0.0))

    @pl.when((k == 0) | (k == _K // _B))
    def _head_block():
        ex = jnp.exp(logits[0:_E] - mx[0:_E])
        p0 = ex[:, 0:1] / jnp.sum(ex, axis=1, keepdims=True)
        p0s[pl.ds(_E * b, _E)] = p0
        h = jax.lax.dot_general(x[0:64], fc1_ref[0], _CT,
                                preferred_element_type=jnp.float32)
        h = h + fc1b_ref[0]
        taps = jnp.transpose(dw_ref[0])
        r0, r1 = h[0:32], h[32:64]
        conv = (_shift_down(r0) * taps[3:4] + r0 * taps[4:5]
                + _shift_up(r0) * taps[5:6]
                + _shift_down(r1) * taps[6:7] + r1 * taps[7:8]
                + _shift_up(r1) * taps[8:9]) + dwb_ref[0]
        g = conv[0:_E]
        g = 0.5 * g * (1.0 + jax.lax.erf(g * 0.7071067811865476))
        gsc[pl.ds(_E * b, _E), :] = g

    @pl.when(k == _K - 1)
    def _final():
        for i in range(_K):
            pltpu.make_async_copy(zblk, out_ref.at[pl.ds(i * _BK, _BK), :],
                                  sem.at[i]).wait()
        prescol = jnp.concatenate(
            [_row_to_col(pres[0:1]), _row_to_col(pres[1:2])], axis=0)
        masked = p0s[...] * prescol
        d8 = masked[0:_E] + masked[_E:2 * _E] + 1e-6
        gs_col = masked / jnp.concatenate([d8, d8], axis=0) * float(_B)
        y = jax.lax.dot_general(gsc[...], fc2_ref[0], _CT,
                                preferred_element_type=jnp.float32)
        ys[...] = (y + fc2b_ref[0]) * gs_col
        cp0 = pltpu.make_async_copy(ys.at[pl.ds(0, _E), :],
                                    out_ref.at[pl.ds(0, _E), :], sem.at[_K])
        cp1 = pltpu.make_async_copy(ys.at[pl.ds(_E, _E), :],
                                    out_ref.at[pl.ds(_N, _E), :],
                                    sem.at[_K + 1])
        cp0.start()
        cp1.start()
        cp0.wait()
        cp1.wait()


def kernel(x, H, W, wg_w, wg_b, fc1_w, fc1_b, dw_w, dw_b, fc2_w, fc2_b):
    xf = x.reshape(_NT, _DIM)
    wgb = wg_b.reshape(1, _E)
    fc1b = fc1_b.reshape(_E, 1, _HID)
    dwf = dw_w.reshape(_E, _HID, 9)
    dwb = dw_b.reshape(_E, 1, _HID)
    fc2b = fc2_b.reshape(_E, 1, _OUT)
    out = pl.pallas_call(
        _moe_kernel,
        grid=(_K,),
        in_specs=[
            pl.BlockSpec((_BK, _DIM), lambda k: (k, 0)),
            pl.BlockSpec((_E, _DIM), lambda k: (0, 0)),
            pl.BlockSpec((1, _E), lambda k: (0, 0)),
            pl.BlockSpec((1, _HID, _DIM), lambda k: (0, 0, 0)),
            pl.BlockSpec((1, 1, _HID), lambda k: (0, 0, 0)),
            pl.BlockSpec((1, _HID, 9), lambda k: (0, 0, 0)),
            pl.BlockSpec((1, 1, _HID), lambda k: (0, 0, 0)),
            pl.BlockSpec((1, _OUT, _HID), lambda k: (0, 0, 0)),
            pl.BlockSpec((1, 1, _OUT), lambda k: (0, 0, 0)),
        ],
        out_specs=pl.BlockSpec(memory_space=pltpu.MemorySpace.HBM),
        out_shape=jax.ShapeDtypeStruct((_NT, _OUT), jnp.float32),
        scratch_shapes=[
            pltpu.VMEM((_BK, _OUT), jnp.float32),
            pltpu.VMEM((_B, _E), jnp.float32),
            pltpu.VMEM((2 * _E, 1), jnp.float32),
            pltpu.VMEM((2 * _E, _HID), jnp.float32),
            pltpu.VMEM((2 * _E, _OUT), jnp.float32),
            pltpu.SemaphoreType.DMA((_K + 2,)),
        ],
    )(xf, wg_w, wgb, fc1_w, fc1b, dwf, dwb, fc2_w, fc2b)
    return (out.reshape(_B, _N, _OUT), None)


# speedup vs baseline: 1.2051x; 1.2051x over previous
import jax
import jax.numpy as jnp
from jax.experimental import pallas as pl
from jax.experimental.pallas import tpu as pltpu

_E = 8
_DIM = 96
_HID = 384
_OUT = 96
_B = 2
_N = 1024
_NT = _B * _N

_CT = (((1,), (1,)), ((), ()))


def _row_to_col(row):
    i = jax.lax.broadcasted_iota(jnp.int32, (_E, _E), 0)
    j = jax.lax.broadcasted_iota(jnp.int32, (_E, _E), 1)
    b = jnp.broadcast_to(row, (_E, _E))
    return jnp.sum(jnp.where(i == j, b, 0.0), axis=1, keepdims=True)


def _shift_down(a):
    return jnp.concatenate([jnp.zeros((1, _HID), jnp.float32), a[:-1]], axis=0)


def _shift_up(a):
    return jnp.concatenate([a[1:], jnp.zeros((1, _HID), jnp.float32)], axis=0)


def _moe_kernel(x_ref, wg_ref, wgb_ref, fc1_ref, fc1b_ref, dw_ref,
                dwb_ref, fc2_ref, fc2b_ref, out_ref):
    x = x_ref[...]
    lt = jax.lax.dot_general(wg_ref[...], x, _CT,
                             preferred_element_type=jnp.float32)
    lt = lt + wgb_ref[...]
    mx = jnp.max(lt, axis=0, keepdims=True)
    iota_s = jax.lax.broadcasted_iota(jnp.int32, (_E, _NT), 0)
    idx = jnp.min(jnp.where(lt == mx, iota_s, _E), axis=0, keepdims=True)
    onehot = jnp.where(iota_s == idx, 1.0, 0.0)
    pres0 = jnp.max(onehot[:, :_N], axis=1, keepdims=True)
    pres1 = jnp.max(onehot[:, _N:], axis=1, keepdims=True)
    e0 = jnp.exp(lt[:, 0:_E] - mx[:, 0:_E])
    e1 = jnp.exp(lt[:, _N:_N + _E] - mx[:, _N:_N + _E])
    p0r0 = e0[0:1] / jnp.sum(e0, axis=0, keepdims=True)
    p0r1 = e1[0:1] / jnp.sum(e1, axis=0, keepdims=True)
    masked0 = _row_to_col(p0r0) * pres0
    masked1 = _row_to_col(p0r1) * pres1
    denom = masked0 + masked1 + 1e-6
    gs_col = jnp.concatenate([masked0 / denom, masked1 / denom],
                             axis=0) * float(_B)
    x64 = jnp.concatenate([x[0:64], x[_N:_N + 64]], axis=0)
    h = jax.lax.dot_general(x64, fc1_ref[0], _CT,
                            preferred_element_type=jnp.float32)
    h = h + fc1b_ref[0]
    taps = jnp.transpose(dw_ref[0])
    outs = []
    for b in range(_B):
        r0 = h[b * 64:b * 64 + 32]
        r1 = h[b * 64 + 32:b * 64 + 64]
        conv = (_shift_down(r0) * taps[3:4] + r0 * taps[4:5]
                + _shift_up(r0) * taps[5:6]
                + _shift_down(r1) * taps[6:7] + r1 * taps[7:8]
                + _shift_up(r1) * taps[8:9]) + dwb_ref[0]
        outs.append(conv[0:_E])
    g = jnp.concatenate(outs, axis=0)
    g = 0.5 * g * (1.0 + jax.lax.erf(g * 0.7071067811865476))
    y = jax.lax.dot_general(g, fc2_ref[0], _CT,
                            preferred_element_type=jnp.float32)
    y = (y + fc2b_ref[0]) * gs_col
    out_ref[...] = jnp.zeros((_NT, _OUT), jnp.float32)
    out_ref[0:_E, :] = y[0:_E]
    out_ref[_N:_N + _E, :] = y[_E:2 * _E]


def kernel(x, H, W, wg_w, wg_b, fc1_w, fc1_b, dw_w, dw_b, fc2_w, fc2_b):
    xf = x.reshape(_NT, _DIM)
    wgb = wg_b.reshape(_E, 1)
    fc1b = fc1_b.reshape(_E, 1, _HID)
    dwf = dw_w.reshape(_E, _HID, 9)
    dwb = dw_b.reshape(_E, 1, _HID)
    fc2b = fc2_b.reshape(_E, 1, _OUT)
    out = pl.pallas_call(
        _moe_kernel,
        grid=(1,),
        in_specs=[
            pl.BlockSpec((_NT, _DIM), lambda i: (0, 0)),
            pl.BlockSpec((_E, _DIM), lambda i: (0, 0)),
            pl.BlockSpec((_E, 1), lambda i: (0, 0)),
            pl.BlockSpec((1, _HID, _DIM), lambda i: (0, 0, 0)),
            pl.BlockSpec((1, 1, _HID), lambda i: (0, 0, 0)),
            pl.BlockSpec((1, _HID, 9), lambda i: (0, 0, 0)),
            pl.BlockSpec((1, 1, _HID), lambda i: (0, 0, 0)),
            pl.BlockSpec((1, _OUT, _HID), lambda i: (0, 0, 0)),
            pl.BlockSpec((1, 1, _OUT), lambda i: (0, 0, 0)),
        ],
        out_specs=pl.BlockSpec((_NT, _OUT), lambda i: (0, 0)),
        out_shape=jax.ShapeDtypeStruct((_NT, _OUT), jnp.float32),
    )(xf, wg_w, wgb, fc1_w, fc1b, dwf, dwb, fc2_w, fc2b)
    return (out.reshape(_B, _N, _OUT), None)


# speedup vs baseline: 1.3273x; 1.1015x over previous
import jax
import jax.numpy as jnp
from jax.experimental import pallas as pl
from jax.experimental.pallas import tpu as pltpu

_E = 8
_DIM = 96
_HID = 384
_OUT = 96
_B = 2
_N = 1024
_NT = _B * _N

_CT = (((1,), (1,)), ((), ()))


def _row_to_col(row):
    i = jax.lax.broadcasted_iota(jnp.int32, (_E, _E), 0)
    j = jax.lax.broadcasted_iota(jnp.int32, (_E, _E), 1)
    b = jnp.broadcast_to(row, (_E, _E))
    return jnp.sum(jnp.where(i == j, b, 0.0), axis=1, keepdims=True)


def _shift_down(a):
    return jnp.concatenate([jnp.zeros((1, _HID), jnp.float32), a[:-1]], axis=0)


def _shift_up(a):
    return jnp.concatenate([a[1:], jnp.zeros((1, _HID), jnp.float32)], axis=0)


def _moe_kernel(x_ref, pp_ref, fc1_ref, fc2_ref, out_ref):
    x = x_ref[...]
    wg = pp_ref[0:_E, 0:_DIM]
    wgb = _row_to_col(pp_ref[8:9, 0:_E])
    fc1b = pp_ref[9:10, :]
    dwb = pp_ref[10:11, :]
    fc2b = pp_ref[11:12, 0:_OUT]
    taps = pp_ref[12:18, :]
    lt = jax.lax.dot_general(wg, x, _CT, preferred_element_type=jnp.float32)
    lt = lt + wgb
    mx = jnp.max(lt, axis=0, keepdims=True)
    iota_s = jax.lax.broadcasted_iota(jnp.int32, (_E, _NT), 0)
    idx = jnp.min(jnp.where(lt == mx, iota_s, _E), axis=0, keepdims=True)
    onehot = jnp.where(iota_s == idx, 1.0, 0.0)
    pres0 = jnp.max(onehot[:, :_N], axis=1, keepdims=True)
    pres1 = jnp.max(onehot[:, _N:], axis=1, keepdims=True)
    e0 = jnp.exp(lt[:, 0:_E] - mx[:, 0:_E])
    e1 = jnp.exp(lt[:, _N:_N + _E] - mx[:, _N:_N + _E])
    p0r0 = e0[0:1] / jnp.sum(e0, axis=0, keepdims=True)
    p0r1 = e1[0:1] / jnp.sum(e1, axis=0, keepdims=True)
    masked0 = _row_to_col(p0r0) * pres0
    masked1 = _row_to_col(p0r1) * pres1
    denom = masked0 + masked1 + 1e-6
    gs_col = jnp.concatenate([masked0 / denom, masked1 / denom],
                             axis=0) * float(_B)
    x64 = jnp.concatenate([x[0:64], x[_N:_N + 64]], axis=0)
    h = jax.lax.dot_general(x64, fc1_ref[0], _CT,
                            preferred_element_type=jnp.float32)
    h = h + fc1b
    outs = []
    for b in range(_B):
        r0 = h[b * 64:b * 64 + 32]
        r1 = h[b * 64 + 32:b * 64 + 64]
        conv = (_shift_down(r0) * taps[0:1] + r0 * taps[1:2]
                + _shift_up(r0) * taps[2:3]
                + _shift_down(r1) * taps[3:4] + r1 * taps[4:5]
                + _shift_up(r1) * taps[5:6]) + dwb
        outs.append(conv[0:_E])
    g = jnp.concatenate(outs, axis=0)
    g = 0.5 * g * (1.0 + jax.lax.erf(g * 0.7071067811865476))
    y = jax.lax.dot_general(g, fc2_ref[0], _CT,
                            preferred_element_type=jnp.float32)
    y = (y + fc2b) * gs_col
    out_ref[...] = jnp.zeros((_NT, _OUT), jnp.float32)
    out_ref[0:_E, :] = y[0:_E]
    out_ref[_N:_N + _E, :] = y[_E:2 * _E]


def kernel(x, H, W, wg_w, wg_b, fc1_w, fc1_b, dw_w, dw_b, fc2_w, fc2_b):
    xf = x.reshape(_NT, _DIM)
    taps = dw_w[0, :, 0].reshape(_HID, 9).T[3:9]
    pp = jnp.zeros((24, _HID), jnp.float32)
    pp = pp.at[0:_E, 0:_DIM].set(wg_w)
    pp = pp.at[8, 0:_E].set(wg_b)
    pp = pp.at[9, :].set(fc1_b[0])
    pp = pp.at[10, :].set(dw_b[0])
    pp = pp.at[11, 0:_OUT].set(fc2_b[0])
    pp = pp.at[12:18, :].set(taps)
    out = pl.pallas_call(
        _moe_kernel,
        grid=(1,),
        in_specs=[
            pl.BlockSpec((_NT, _DIM), lambda i: (0, 0)),
            pl.BlockSpec((24, _HID), lambda i: (0, 0)),
            pl.BlockSpec((1, _HID, _DIM), lambda i: (0, 0, 0)),
            pl.BlockSpec((1, _OUT, _HID), lambda i: (0, 0, 0)),
        ],
        out_specs=pl.BlockSpec((_NT, _OUT), lambda i: (0, 0)),
        out_shape=jax.ShapeDtypeStruct((_NT, _OUT), jnp.float32),
    )(xf, pp, fc1_w, fc2_w)
    return (out.reshape(_B, _N, _OUT), None)
